# Initial kernel scaffold; baseline (speedup 1.0000x reference)
import jax
import jax.numpy as jnp
from jax.experimental import pallas as pl
from jax.experimental.pallas import tpu as pltpu


def _probe_body(x_ref, o_ref):
    q2 = x_ref[0]
    q3 = q2.reshape(64, 128, 128)
    F = jnp.full((64, 64), 0.5, jnp.float32)
    Y = jax.lax.dot_general(
        F, q3, (((1,), (0,)), ((), ())),
        precision=jax.lax.Precision.HIGHEST,
        preferred_element_type=jnp.float32,
    )
    y2 = jnp.sum(Y, axis=0)

    d = (q2[0, 0].astype(jnp.int32)) & 4095
    r1 = pltpu.roll(q2, d, axis=0)
    r2 = pltpu.roll(q2, d, axis=1)
    l1 = x_ref[0, pl.ds(d, 4096), :]

    o_ref[0] = r1 + r2
    o_ref[0, :4096, :] += l1
    o_ref[0, :128, :] += y2


def kernel(x, W, b):
    out = pl.pallas_call(
        _probe_body,
        grid=(4,),
        in_specs=[pl.BlockSpec((1, 8192, 128), lambda i: (i, 0, 0))],
        out_specs=pl.BlockSpec((1, 8192, 128), lambda i: (i, 0, 0)),
        out_shape=jax.ShapeDtypeStruct((4, 8192, 128), jnp.float32),
    )(x[:, :, :128])
    o = jnp.zeros_like(x)
    o = o.at[:, :, :128].set(out)
    return o


# speedup vs baseline: 1.9409x; 1.9409x over previous
import functools
import math

import numpy as np
import jax
import jax.numpy as jnp
from jax.experimental import pallas as pl
from jax.experimental.pallas import tpu as pltpu

B, T, D = 4, 8192, 768
N1, N2 = 64, 128
DP = D // 2
TOPK = 13
HI = jax.lax.Precision.HIGHEST


def _consts():
    t1 = np.arange(N1)
    t2 = np.arange(N2)
    e64 = np.exp(-2j * np.pi * np.outer(t1, t1) / N1)
    tw = np.exp(-2j * np.pi * np.outer(t2, t1) / T)[:, None, :]
    f128 = np.exp(-2j * np.pi * np.outer(t2, t2) / N2)
    ic = np.exp(+2j * np.pi * np.outer(t2, t2) / N2)
    itw = np.exp(+2j * np.pi * np.outer(t1, t2) / T)
    i64 = np.exp(+2j * np.pi * np.outer(t1, t1) / N1)
    rev64 = np.eye(N1, dtype=np.float32)[::-1]
    rev128 = np.eye(N2, dtype=np.float32)[::-1]
    as_f32 = lambda a: jnp.asarray(np.ascontiguousarray(a), jnp.float32)
    return {
        "e64r": as_f32(e64.real), "e64i": as_f32(e64.imag),
        "twr": as_f32(tw.real), "twi": as_f32(tw.imag),
        "f128r": as_f32(f128.real), "f128i": as_f32(f128.imag),
        "icr": as_f32(ic.real), "ici": as_f32(ic.imag),
        "itwr": as_f32(itw.real), "itwi": as_f32(itw.imag),
        "i64r": as_f32(i64.real), "i64i": as_f32(i64.imag),
        "rev64": as_f32(rev64), "rev128": as_f32(rev128),
    }


def _proj_body(x_ref, wt_ref, b_ref, o_ref):
    o_ref[...] = (
        jax.lax.dot_general(
            x_ref[...], wt_ref[...], (((1,), (0,)), ((), ())),
            precision=HI, preferred_element_type=jnp.float32)
        + b_ref[...]
    )


def _projection(x, W, b):
    x2 = x.reshape(B * T, D)
    wt = W.T
    b2 = b.reshape(1, D)
    rows = 512
    q2 = pl.pallas_call(
        _proj_body,
        grid=(B * T // rows,),
        in_specs=[
            pl.BlockSpec((rows, D), lambda i: (i, 0)),
            pl.BlockSpec((D, D), lambda i: (0, 0)),
            pl.BlockSpec((1, D), lambda i: (0, 0)),
        ],
        out_specs=pl.BlockSpec((rows, D), lambda i: (i, 0)),
        out_shape=jax.ShapeDtypeStruct((B * T, D), jnp.float32),
    )(x2, wt, b2)
    return q2.reshape(B, T, D)


def _cmul3(ar, ai, br, bi):
    return ar * br - ai * bi, ar * bi + ai * br


def _dot3_rhs(m, z):
    return jax.lax.dot_general(
        m, z, (((1,), (0,)), ((), ())), precision=HI,
        preferred_element_type=jnp.float32)


def _dot3_lhs(z, m):
    return jax.lax.dot_general(
        z, m, (((0,), (0,)), ((), ())), precision=HI,
        preferred_element_type=jnp.float32)


_TC = 16
_NK = N2 // _TC
_DC = 128
_NJ = DP // _DC


def _fft_body(zr_ref, zi_ref, e64r, e64i, twr, twi, f128r, f128i, s_ref,
              xr_s, xi_s):
    j = pl.program_id(1)
    k = pl.program_id(2)
    z3r = zr_ref[0]
    z3i = zi_ref[0]
    yr = _dot3_lhs(z3r, e64r[...]) - _dot3_lhs(z3i, e64i[...])
    yi = _dot3_lhs(z3r, e64i[...]) + _dot3_lhs(z3i, e64r[...])
    zr2, zi2 = _cmul3(yr, yi, twr[...], twi[...])
    @pl.when(k == 0)
    def _():
        xr_s[...] = jnp.zeros_like(xr_s)
        xi_s[...] = jnp.zeros_like(xi_s)

    xr_s[...] += _dot3_lhs(zr2, f128r[...])
    xr_s[...] -= _dot3_lhs(zi2, f128i[...])
    xi_s[...] += _dot3_lhs(zr2, f128i[...])
    xi_s[...] += _dot3_lhs(zi2, f128r[...])

    @pl.when(k == _NK - 1)
    def _():
        xr = xr_s[...]
        xi = xi_s[...]
        sp = jnp.sum(xr * xr + xi * xi, axis=0)

        @pl.when(j == 0)
        def _():
            s_ref[0] = sp

        @pl.when(j != 0)
        def _():
            s_ref[0] += sp


def _power_spectrum(q, c):
    q4 = q.reshape(B, N1, N2, D)
    return pl.pallas_call(
        _fft_body,
        grid=(B, _NJ, _NK),
        in_specs=[
            pl.BlockSpec((1, N1, _TC, _DC), lambda b, j, k: (b, 0, k, j)),
            pl.BlockSpec((1, N1, _TC, _DC),
                         lambda b, j, k: (b, 0, k, j + _NJ)),
            pl.BlockSpec((N1, N1), lambda b, j, k: (0, 0)),
            pl.BlockSpec((N1, N1), lambda b, j, k: (0, 0)),
            pl.BlockSpec((_TC, 1, N1), lambda b, j, k: (k, 0, 0)),
            pl.BlockSpec((_TC, 1, N1), lambda b, j, k: (k, 0, 0)),
            pl.BlockSpec((_TC, N2), lambda b, j, k: (k, 0)),
            pl.BlockSpec((_TC, N2), lambda b, j, k: (k, 0)),
        ],
        out_specs=pl.BlockSpec((1, N1, N2), lambda b, j, k: (b, 0, 0)),
        out_shape=jax.ShapeDtypeStruct((B, N1, N2), jnp.float32),
        scratch_shapes=[
            pltpu.VMEM((_DC, N1, N2), jnp.float32),
            pltpu.VMEM((_DC, N1, N2), jnp.float32),
        ],
    )(q4, q4, c["e64r"], c["e64i"], c["twr"], c["twi"],
      c["f128r"], c["f128i"])


def _topk_body(s_ref, icr, ici, itwr, itwi, i64r, i64i, r64, r128, d_ref):
    s = s_ref[0]
    gr = jnp.dot(s, icr[...], precision=HI)
    gi = jnp.dot(s, ici[...], precision=HI)
    hr, hi = _cmul3(gr, gi, itwr[...], itwi[...])
    corr = (jnp.dot(i64r[...], hr, precision=HI)
            - jnp.dot(i64i[...], hi, precision=HI))
    m1 = jnp.dot(jnp.dot(r64[...], corr, precision=HI), r128[...],
                 precision=HI)
    r = jnp.roll(m1, 1, axis=1)
    colidx = jax.lax.broadcasted_iota(jnp.int32, (N1, N2), 1)
    mirror = jnp.where(colidx == 0, jnp.roll(r, 1, axis=0), r)
    flat = (jax.lax.broadcasted_iota(jnp.int32, (N1, N2), 0) * N2
            + jax.lax.broadcasted_iota(jnp.int32, (N1, N2), 1))
    neg = jnp.float32(-jnp.inf)
    c = corr + mirror
    for i in range(TOPK):
        m = jnp.max(c)
        idx = jnp.min(jnp.where(c == m, flat, T))
        d_ref[0, 0, i] = idx
        c = jnp.where(flat == idx, neg, c)
    for i in range(TOPK, 16):
        d_ref[0, 0, i] = 0


def _top_delays(s, c):
    return pl.pallas_call(
        _topk_body,
        grid=(B,),
        in_specs=[
            pl.BlockSpec((1, N1, N2), lambda b: (b, 0, 0)),
            pl.BlockSpec((N2, N2), lambda b: (0, 0)),
            pl.BlockSpec((N2, N2), lambda b: (0, 0)),
            pl.BlockSpec((N1, N2), lambda b: (0, 0)),
            pl.BlockSpec((N1, N2), lambda b: (0, 0)),
            pl.BlockSpec((N1, N1), lambda b: (0, 0)),
            pl.BlockSpec((N1, N1), lambda b: (0, 0)),
            pl.BlockSpec((N1, N1), lambda b: (0, 0)),
            pl.BlockSpec((N2, N2), lambda b: (0, 0)),
        ],
        out_specs=pl.BlockSpec(
            (1, 1, 16), lambda b: (b, 0, 0), memory_space=pltpu.SMEM),
        out_shape=jax.ShapeDtypeStruct((B, 1, 16), jnp.int32),
    )(s, c["icr"], c["ici"], c["itwr"], c["itwi"], c["i64r"], c["i64i"],
      c["rev64"], c["rev128"])


def _combine_body(d_ref, qe_ref, v_ref, o_ref):
    b = pl.program_id(0)
    acc = qe_ref[0, pl.ds(d_ref[b, 0], T), :]
    for i in range(1, TOPK):
        acc = acc + qe_ref[0, pl.ds(d_ref[b, i], T), :]
    o_ref[0] = v_ref[0] + acc * jnp.float32(1.0 / TOPK)


def _combine(delays, q_ext, x):
    dc = 128
    return pl.pallas_call(
        _combine_body,
        grid=(B, D // dc),
        in_specs=[
            pl.BlockSpec((B, 16), lambda b, j: (0, 0),
                         memory_space=pltpu.SMEM),
            pl.BlockSpec((1, 2 * T, dc), lambda b, j: (b, 0, j)),
            pl.BlockSpec((1, T, dc), lambda b, j: (b, 0, j)),
        ],
        out_specs=pl.BlockSpec((1, T, dc), lambda b, j: (b, 0, j)),
        out_shape=jax.ShapeDtypeStruct((B, T, D), jnp.float32),
    )(delays, q_ext, x)


def kernel(x, W, b):
    c = _consts()
    q = _projection(x, W, b)
    s = _power_spectrum(q, c)
    delays = _top_delays(s, c).reshape(B, 16)
    q_ext = jnp.concatenate([q, q], axis=1)
    return _combine(delays, q_ext, x)


# speedup vs baseline: 4.3270x; 2.2294x over previous
import functools
import math

import numpy as np
import jax
import jax.numpy as jnp
from jax.experimental import pallas as pl
from jax.experimental.pallas import tpu as pltpu

B, T, D = 4, 8192, 768
N1, N2 = 64, 128
DP = D // 2
TOPK = 13
HI = jax.lax.Precision.HIGHEST

_TC = 16
_NK = N2 // _TC
_MC = _TC * DP


def _consts():
    t1 = np.arange(N1)
    t2 = np.arange(N2)
    e64 = np.exp(-2j * np.pi * np.outer(t1, t1) / N1)
    tw = np.exp(-2j * np.pi * np.outer(t1, t2) / T)[:, :, None]
    f128 = np.exp(-2j * np.pi * np.outer(t2, t2) / N2)
    ic = np.exp(+2j * np.pi * np.outer(t2, t2) / N2)
    itw = np.exp(+2j * np.pi * np.outer(t1, t2) / T)
    i64 = np.exp(+2j * np.pi * np.outer(t1, t1) / N1)
    rev64 = np.eye(N1, dtype=np.float32)[::-1]
    rev128 = np.eye(N2, dtype=np.float32)[::-1]
    as_f32 = lambda a: jnp.asarray(np.ascontiguousarray(a), jnp.float32)
    return {
        "e64r": as_f32(e64.real), "e64i": as_f32(e64.imag),
        "twr": as_f32(tw.real), "twi": as_f32(tw.imag),
        "f128r": as_f32(f128.real), "f128i": as_f32(f128.imag),
        "icr": as_f32(ic.real), "ici": as_f32(ic.imag),
        "itwr": as_f32(itw.real), "itwi": as_f32(itw.imag),
        "i64r": as_f32(i64.real), "i64i": as_f32(i64.imag),
        "rev64": as_f32(rev64), "rev128": as_f32(rev128),
    }


def _dot(a, bm):
    return jax.lax.dot_general(
        a, bm, (((1,), (0,)), ((), ())), precision=HI,
        preferred_element_type=jnp.float32)


def _proj_body(x_ref, wt_ref, b_ref, oa_ref, ob_ref):
    res = _dot(x_ref[...], wt_ref[...]) + b_ref[...]
    ra = res[:, :DP]
    rb = res[:, DP:]
    oa_ref[0, 0] = ra
    oa_ref[0, 1] = ra
    ob_ref[0, 0] = rb
    ob_ref[0, 1] = rb


def _projection(x, W, b):
    x2 = x.reshape(B * T, D)
    wt = W.T
    b2 = b.reshape(1, D)
    rows = 512
    nb = T // rows
    qa, qb = pl.pallas_call(
        _proj_body,
        grid=(B, nb),
        in_specs=[
            pl.BlockSpec(
                (rows, D), lambda bb, i, _nb=nb: (bb * _nb + i, 0)),
            pl.BlockSpec((D, D), lambda bb, i: (0, 0)),
            pl.BlockSpec((1, D), lambda bb, i: (0, 0)),
        ],
        out_specs=[
            pl.BlockSpec((1, 2, rows, DP), lambda bb, i: (bb, 0, i, 0)),
            pl.BlockSpec((1, 2, rows, DP), lambda bb, i: (bb, 0, i, 0)),
        ],
        out_shape=[
            jax.ShapeDtypeStruct((B, 2, T, DP), jnp.float32),
            jax.ShapeDtypeStruct((B, 2, T, DP), jnp.float32),
        ],
    )(x2, wt, b2)
    return qa.reshape(B, 2 * T, DP), qb.reshape(B, 2 * T, DP)


def _fft1_body(zr_ref, zi_ref, e64r, e64i, yr_ref, yi_ref):
    zr = zr_ref[0]
    zi = zi_ref[0]
    yr_ref[0, 0] = _dot(e64r[...], zr) - _dot(e64i[...], zi)
    yi_ref[0, 0] = _dot(e64r[...], zi) + _dot(e64i[...], zr)


def _fft_stage1(qa_ext, qb_ext, c):
    za = qa_ext.reshape(B, 2 * N1, N2 * DP)
    zb = qb_ext.reshape(B, 2 * N1, N2 * DP)
    return pl.pallas_call(
        _fft1_body,
        grid=(B, _NK),
        in_specs=[
            pl.BlockSpec((1, N1, _MC), lambda b, k: (b, 0, k)),
            pl.BlockSpec((1, N1, _MC), lambda b, k: (b, 0, k)),
            pl.BlockSpec((N1, N1), lambda b, k: (0, 0)),
            pl.BlockSpec((N1, N1), lambda b, k: (0, 0)),
        ],
        out_specs=[
            pl.BlockSpec((1, 1, N1, _MC), lambda b, k: (b, k, 0, 0)),
            pl.BlockSpec((1, 1, N1, _MC), lambda b, k: (b, k, 0, 0)),
        ],
        out_shape=[
            jax.ShapeDtypeStruct((B, _NK, N1, _MC), jnp.float32),
            jax.ShapeDtypeStruct((B, _NK, N1, _MC), jnp.float32),
        ],
    )(za, zb, c["e64r"], c["e64i"])


def _fft2_body(yr_ref, yi_ref, twr, twi, f128r, f128i, s_ref):
    f1 = pl.program_id(1)
    yr = yr_ref[0, :, 0].reshape(N2, DP)
    yi = yi_ref[0, :, 0].reshape(N2, DP)
    wr = twr[0]
    wi = twi[0]
    zr = yr * wr - yi * wi
    zi = yr * wi + yi * wr
    xr = _dot(f128r[...], zr) - _dot(f128i[...], zi)
    xi = _dot(f128r[...], zi) + _dot(f128i[...], zr)
    s_ref[0, f1, :] = jnp.sum(xr * xr + xi * xi, axis=1)


def _fft_stage2(yr4, yi4, c):
    yr5 = yr4.reshape(B, _NK, N1, _TC, DP)
    yi5 = yi4.reshape(B, _NK, N1, _TC, DP)
    return pl.pallas_call(
        _fft2_body,
        grid=(B, N1),
        in_specs=[
            pl.BlockSpec((1, _NK, 1, _TC, DP),
                         lambda b, f: (b, 0, f, 0, 0)),
            pl.BlockSpec((1, _NK, 1, _TC, DP),
                         lambda b, f: (b, 0, f, 0, 0)),
            pl.BlockSpec((1, N2, 1), lambda b, f: (f, 0, 0)),
            pl.BlockSpec((1, N2, 1), lambda b, f: (f, 0, 0)),
            pl.BlockSpec((N2, N2), lambda b, f: (0, 0)),
            pl.BlockSpec((N2, N2), lambda b, f: (0, 0)),
        ],
        out_specs=pl.BlockSpec((1, N1, N2), lambda b, f: (b, 0, 0)),
        out_shape=jax.ShapeDtypeStruct((B, N1, N2), jnp.float32),
    )(yr5, yi5, c["twr"], c["twi"], c["f128r"], c["f128i"])


def _cmul(ar, ai, br, bi):
    return ar * br - ai * bi, ar * bi + ai * br


def _topk_body(s_ref, icr, ici, itwr, itwi, i64r, i64i, r64, r128, d_ref):
    s = s_ref[0]
    gr = jnp.dot(s, icr[...], precision=HI)
    gi = jnp.dot(s, ici[...], precision=HI)
    hr, hi = _cmul(gr, gi, itwr[...], itwi[...])
    corr = (jnp.dot(i64r[...], hr, precision=HI)
            - jnp.dot(i64i[...], hi, precision=HI))
    m1 = jnp.dot(jnp.dot(r64[...], corr, precision=HI), r128[...],
                 precision=HI)
    r = jnp.roll(m1, 1, axis=1)
    colidx = jax.lax.broadcasted_iota(jnp.int32, (N1, N2), 1)
    mirror = jnp.where(colidx == 0, jnp.roll(r, 1, axis=0), r)
    flat = (jax.lax.broadcasted_iota(jnp.int32, (N1, N2), 0) * N2
            + jax.lax.broadcasted_iota(jnp.int32, (N1, N2), 1))
    neg = jnp.float32(-jnp.inf)
    c = corr + mirror
    for i in range(TOPK):
        m = jnp.max(c)
        idx = jnp.min(jnp.where(c == m, flat, T))
        d_ref[0, 0, i] = idx
        c = jnp.where(flat == idx, neg, c)
    for i in range(TOPK, 16):
        d_ref[0, 0, i] = 0


def _top_delays(s, c):
    return pl.pallas_call(
        _topk_body,
        grid=(B,),
        in_specs=[
            pl.BlockSpec((1, N1, N2), lambda b: (b, 0, 0)),
            pl.BlockSpec((N2, N2), lambda b: (0, 0)),
            pl.BlockSpec((N2, N2), lambda b: (0, 0)),
            pl.BlockSpec((N1, N2), lambda b: (0, 0)),
            pl.BlockSpec((N1, N2), lambda b: (0, 0)),
            pl.BlockSpec((N1, N1), lambda b: (0, 0)),
            pl.BlockSpec((N1, N1), lambda b: (0, 0)),
            pl.BlockSpec((N1, N1), lambda b: (0, 0)),
            pl.BlockSpec((N2, N2), lambda b: (0, 0)),
        ],
        out_specs=pl.BlockSpec(
            (1, 1, 16), lambda b: (b, 0, 0), memory_space=pltpu.SMEM),
        out_shape=jax.ShapeDtypeStruct((B, 1, 16), jnp.int32),
    )(s, c["icr"], c["ici"], c["itwr"], c["itwi"], c["i64r"], c["i64i"],
      c["rev64"], c["rev128"])


def _combine_body(d_ref, qa_ref, qb_ref, v_ref, o_ref):
    b = pl.program_id(0)
    h = pl.program_id(2)
    inv = jnp.float32(1.0 / TOPK)

    @pl.when(h == 0)
    def _():
        acc = qa_ref[0, pl.ds(d_ref[b, 0], T), :]
        for i in range(1, TOPK):
            acc = acc + qa_ref[0, pl.ds(d_ref[b, i], T), :]
        o_ref[0] = v_ref[0] + acc * inv

    @pl.when(h == 1)
    def _():
        acc = qb_ref[0, pl.ds(d_ref[b, 0], T), :]
        for i in range(1, TOPK):
            acc = acc + qb_ref[0, pl.ds(d_ref[b, i], T), :]
        o_ref[0] = v_ref[0] + acc * inv


def _combine(delays, qa_ext, qb_ext, x):
    dc = 128
    nj = DP // dc
    return pl.pallas_call(
        _combine_body,
        grid=(B, nj, 2),
        in_specs=[
            pl.BlockSpec((B, 16), lambda b, j, h: (0, 0),
                         memory_space=pltpu.SMEM),
            pl.BlockSpec((1, 2 * T, dc), lambda b, j, h: (b, 0, j)),
            pl.BlockSpec((1, 2 * T, dc), lambda b, j, h: (b, 0, j)),
            pl.BlockSpec((1, T, dc),
                         lambda b, j, h, _nj=nj: (b, 0, h * _nj + j)),
        ],
        out_specs=pl.BlockSpec(
            (1, T, dc), lambda b, j, h, _nj=nj: (b, 0, h * _nj + j)),
        out_shape=jax.ShapeDtypeStruct((B, T, D), jnp.float32),
    )(delays, qa_ext, qb_ext, x)


def kernel(x, W, b):
    c = _consts()
    qa_ext, qb_ext = _projection(x, W, b)
    yr4, yi4 = _fft_stage1(qa_ext, qb_ext, c)
    s = _fft_stage2(yr4, yi4, c)
    delays = _top_delays(s, c).reshape(B, 16)
    return _combine(delays, qa_ext, qb_ext, x)


# speedup vs baseline: 4.5330x; 1.0476x over previous
import functools
import math

import numpy as np
import jax
import jax.numpy as jnp
from jax.experimental import pallas as pl
from jax.experimental.pallas import tpu as pltpu

B, T, D = 4, 8192, 768
N1, N2 = 64, 128
DP = D // 2
TOPK = 13
HI = jax.lax.Precision.HIGHEST

_TC = 16
_NK = N2 // _TC
_MC = _TC * DP


def _consts():
    t1 = np.arange(N1)
    t2 = np.arange(N2)
    e64 = np.exp(-2j * np.pi * np.outer(t1, t1) / N1)
    tw = np.exp(-2j * np.pi * np.outer(t1, t2) / T)[:, :, None]
    f128 = np.exp(-2j * np.pi * np.outer(t2, t2) / N2)
    ic = np.exp(+2j * np.pi * np.outer(t2, t2) / N2)
    itw = np.exp(+2j * np.pi * np.outer(t1, t2) / T)
    i64 = np.exp(+2j * np.pi * np.outer(t1, t1) / N1)
    rev64 = np.eye(N1, dtype=np.float32)[::-1]
    rev128 = np.eye(N2, dtype=np.float32)[::-1]
    as_f32 = lambda a: jnp.asarray(np.ascontiguousarray(a), jnp.float32)
    return {
        "e64r": as_f32(e64.real), "e64i": as_f32(e64.imag),
        "twr": as_f32(tw.real), "twi": as_f32(tw.imag),
        "f128r": as_f32(f128.real), "f128i": as_f32(f128.imag),
        "icr": as_f32(ic.real), "ici": as_f32(ic.imag),
        "itwr": as_f32(itw.real), "itwi": as_f32(itw.imag),
        "i64r": as_f32(i64.real), "i64i": as_f32(i64.imag),
        "rev64": as_f32(rev64), "rev128": as_f32(rev128),
    }


def _dot(a, bm):
    return jax.lax.dot_general(
        a, bm, (((1,), (0,)), ((), ())), precision=HI,
        preferred_element_type=jnp.float32)


def _proj_body(x_ref, wt_ref, b_ref, oa_ref, ob_ref):
    res = _dot(x_ref[...], wt_ref[...]) + b_ref[...]
    ra = res[:, :DP]
    rb = res[:, DP:]
    oa_ref[0, 0] = ra
    oa_ref[0, 1] = ra
    ob_ref[0, 0] = rb
    ob_ref[0, 1] = rb


def _projection(x, W, b):
    x2 = x.reshape(B * T, D)
    wt = W.T
    b2 = b.reshape(1, D)
    rows = 512
    nb = T // rows
    qa, qb = pl.pallas_call(
        _proj_body,
        grid=(B, nb),
        in_specs=[
            pl.BlockSpec(
                (rows, D), lambda bb, i, _nb=nb: (bb * _nb + i, 0)),
            pl.BlockSpec((D, D), lambda bb, i: (0, 0)),
            pl.BlockSpec((1, D), lambda bb, i: (0, 0)),
        ],
        out_specs=[
            pl.BlockSpec((1, 2, rows, DP), lambda bb, i: (bb, 0, i, 0)),
            pl.BlockSpec((1, 2, rows, DP), lambda bb, i: (bb, 0, i, 0)),
        ],
        out_shape=[
            jax.ShapeDtypeStruct((B, 2, T, DP), jnp.float32),
            jax.ShapeDtypeStruct((B, 2, T, DP), jnp.float32),
        ],
    )(x2, wt, b2)
    return qa.reshape(B, 2 * T, DP), qb.reshape(B, 2 * T, DP)


def _fft1_body(zr_ref, zi_ref, e64r, e64i, yr_ref, yi_ref):
    zr = zr_ref[0]
    zi = zi_ref[0]
    yr_ref[0, 0] = _dot(e64r[...], zr) - _dot(e64i[...], zi)
    yi_ref[0, 0] = _dot(e64r[...], zi) + _dot(e64i[...], zr)


def _fft_stage1(qa_ext, qb_ext, c):
    za = qa_ext.reshape(B, 2 * N1, N2 * DP)
    zb = qb_ext.reshape(B, 2 * N1, N2 * DP)
    return pl.pallas_call(
        _fft1_body,
        grid=(B, _NK),
        in_specs=[
            pl.BlockSpec((1, N1, _MC), lambda b, k: (b, 0, k)),
            pl.BlockSpec((1, N1, _MC), lambda b, k: (b, 0, k)),
            pl.BlockSpec((N1, N1), lambda b, k: (0, 0)),
            pl.BlockSpec((N1, N1), lambda b, k: (0, 0)),
        ],
        out_specs=[
            pl.BlockSpec((1, 1, N1, _MC), lambda b, k: (b, k, 0, 0)),
            pl.BlockSpec((1, 1, N1, _MC), lambda b, k: (b, k, 0, 0)),
        ],
        out_shape=[
            jax.ShapeDtypeStruct((B, _NK, N1, _MC), jnp.float32),
            jax.ShapeDtypeStruct((B, _NK, N1, _MC), jnp.float32),
        ],
    )(za, zb, c["e64r"], c["e64i"])


def _fft2_body(yr_ref, yi_ref, twr, twi, f128r, f128i, s_ref):
    g = pl.program_id(1)
    for u in range(2):
        yr = yr_ref[0, :, u].reshape(N2, DP)
        yi = yi_ref[0, :, u].reshape(N2, DP)
        wr = twr[0, u]
        wi = twi[0, u]
        zr = yr * wr - yi * wi
        zi = yr * wi + yi * wr
        xr = _dot(f128r[...], zr) - _dot(f128i[...], zi)
        xi = _dot(f128r[...], zi) + _dot(f128i[...], zr)
        s_ref[0, 2 * g + u, :] = jnp.sum(xr * xr + xi * xi, axis=1)


def _fft_stage2(yr4, yi4, c):
    yr5 = yr4.reshape(B, _NK, N1, _TC, DP)
    yi5 = yi4.reshape(B, _NK, N1, _TC, DP)
    twr = c["twr"].reshape(N1 // 2, 2, N2, 1)
    twi = c["twi"].reshape(N1 // 2, 2, N2, 1)
    return pl.pallas_call(
        _fft2_body,
        grid=(B, N1 // 2),
        in_specs=[
            pl.BlockSpec((1, _NK, 2, _TC, DP),
                         lambda b, g: (b, 0, g, 0, 0)),
            pl.BlockSpec((1, _NK, 2, _TC, DP),
                         lambda b, g: (b, 0, g, 0, 0)),
            pl.BlockSpec((1, 2, N2, 1), lambda b, g: (g, 0, 0, 0)),
            pl.BlockSpec((1, 2, N2, 1), lambda b, g: (g, 0, 0, 0)),
            pl.BlockSpec((N2, N2), lambda b, g: (0, 0)),
            pl.BlockSpec((N2, N2), lambda b, g: (0, 0)),
        ],
        out_specs=pl.BlockSpec((1, N1, N2), lambda b, g: (b, 0, 0)),
        out_shape=jax.ShapeDtypeStruct((B, N1, N2), jnp.float32),
    )(yr5, yi5, twr, twi, c["f128r"], c["f128i"])


def _cmul(ar, ai, br, bi):
    return ar * br - ai * bi, ar * bi + ai * br


def _topk_body(s_ref, icr, ici, itwr, itwi, i64r, i64i, r64, r128, d_ref):
    s = s_ref[0]
    gr = jnp.dot(s, icr[...], precision=HI)
    gi = jnp.dot(s, ici[...], precision=HI)
    hr, hi = _cmul(gr, gi, itwr[...], itwi[...])
    corr = (jnp.dot(i64r[...], hr, precision=HI)
            - jnp.dot(i64i[...], hi, precision=HI))
    m1 = jnp.dot(jnp.dot(r64[...], corr, precision=HI), r128[...],
                 precision=HI)
    r = jnp.roll(m1, 1, axis=1)
    colidx = jax.lax.broadcasted_iota(jnp.int32, (N1, N2), 1)
    mirror = jnp.where(colidx == 0, jnp.roll(r, 1, axis=0), r)
    flat = (jax.lax.broadcasted_iota(jnp.int32, (N1, N2), 0) * N2
            + jax.lax.broadcasted_iota(jnp.int32, (N1, N2), 1))
    neg = jnp.float32(-jnp.inf)
    c = corr + mirror
    for i in range(TOPK):
        m = jnp.max(c)
        idx = jnp.min(jnp.where(c == m, flat, T))
        d_ref[0, 0, i] = idx
        c = jnp.where(flat == idx, neg, c)
    for i in range(TOPK, 16):
        d_ref[0, 0, i] = 0


def _top_delays(s, c):
    return pl.pallas_call(
        _topk_body,
        grid=(B,),
        in_specs=[
            pl.BlockSpec((1, N1, N2), lambda b: (b, 0, 0)),
            pl.BlockSpec((N2, N2), lambda b: (0, 0)),
            pl.BlockSpec((N2, N2), lambda b: (0, 0)),
            pl.BlockSpec((N1, N2), lambda b: (0, 0)),
            pl.BlockSpec((N1, N2), lambda b: (0, 0)),
            pl.BlockSpec((N1, N1), lambda b: (0, 0)),
            pl.BlockSpec((N1, N1), lambda b: (0, 0)),
            pl.BlockSpec((N1, N1), lambda b: (0, 0)),
            pl.BlockSpec((N2, N2), lambda b: (0, 0)),
        ],
        out_specs=pl.BlockSpec(
            (1, 1, 16), lambda b: (b, 0, 0), memory_space=pltpu.SMEM),
        out_shape=jax.ShapeDtypeStruct((B, 1, 16), jnp.int32),
    )(s, c["icr"], c["ici"], c["itwr"], c["itwi"], c["i64r"], c["i64i"],
      c["rev64"], c["rev128"])


def _combine_body(d_ref, qa_ref, qb_ref, v_ref, o_ref):
    b = pl.program_id(0)
    h = pl.program_id(2)
    inv = jnp.float32(1.0 / TOPK)

    @pl.when(h == 0)
    def _():
        acc = qa_ref[0, pl.ds(d_ref[b, 0], T), :]
        for i in range(1, TOPK):
            acc = acc + qa_ref[0, pl.ds(d_ref[b, i], T), :]
        o_ref[0] = v_ref[0] + acc * inv

    @pl.when(h == 1)
    def _():
        acc = qb_ref[0, pl.ds(d_ref[b, 0], T), :]
        for i in range(1, TOPK):
            acc = acc + qb_ref[0, pl.ds(d_ref[b, i], T), :]
        o_ref[0] = v_ref[0] + acc * inv


def _combine(delays, qa_ext, qb_ext, x):
    dc = 128
    nj = DP // dc
    return pl.pallas_call(
        _combine_body,
        grid=(B, nj, 2),
        in_specs=[
            pl.BlockSpec((B, 16), lambda b, j, h: (0, 0),
                         memory_space=pltpu.SMEM),
            pl.BlockSpec((1, 2 * T, dc), lambda b, j, h: (b, 0, j)),
            pl.BlockSpec((1, 2 * T, dc), lambda b, j, h: (b, 0, j)),
            pl.BlockSpec((1, T, dc),
                         lambda b, j, h, _nj=nj: (b, 0, h * _nj + j)),
        ],
        out_specs=pl.BlockSpec(
            (1, T, dc), lambda b, j, h, _nj=nj: (b, 0, h * _nj + j)),
        out_shape=jax.ShapeDtypeStruct((B, T, D), jnp.float32),
    )(delays, qa_ext, qb_ext, x)


def kernel(x, W, b):
    c = _consts()
    qa_ext, qb_ext = _projection(x, W, b)
    yr4, yi4 = _fft_stage1(qa_ext, qb_ext, c)
    s = _fft_stage2(yr4, yi4, c)
    delays = _top_delays(s, c).reshape(B, 16)
    return _combine(delays, qa_ext, qb_ext, x)


# speedup vs baseline: 4.5588x; 1.0057x over previous
import functools
import math

import numpy as np
import jax
import jax.numpy as jnp
from jax.experimental import pallas as pl
from jax.experimental.pallas import tpu as pltpu

B, T, D = 4, 8192, 768
N1, N2 = 64, 128
DP = D // 2
TOPK = 13
HI = jax.lax.Precision.HIGHEST

_TC = 16
_NK = N2 // _TC
_MC = _TC * DP


def _consts():
    t1 = np.arange(N1)
    t2 = np.arange(N2)
    e64 = np.exp(-2j * np.pi * np.outer(t1, t1) / N1)
    tw = np.exp(-2j * np.pi * np.outer(t1, t2) / T)[:, :, None]
    f128 = np.exp(-2j * np.pi * np.outer(t2, t2) / N2)
    ic = np.exp(+2j * np.pi * np.outer(t2, t2) / N2)
    itw = np.exp(+2j * np.pi * np.outer(t1, t2) / T)
    i64 = np.exp(+2j * np.pi * np.outer(t1, t1) / N1)
    rev64 = np.eye(N1, dtype=np.float32)[::-1]
    rev128 = np.eye(N2, dtype=np.float32)[::-1]
    as_f32 = lambda a: jnp.asarray(np.ascontiguousarray(a), jnp.float32)
    return {
        "e64r": as_f32(e64.real), "e64i": as_f32(e64.imag),
        "twr": as_f32(tw.real), "twi": as_f32(tw.imag),
        "f128r": as_f32(f128.real), "f128i": as_f32(f128.imag),
        "icr": as_f32(ic.real), "ici": as_f32(ic.imag),
        "itwr": as_f32(itw.real), "itwi": as_f32(itw.imag),
        "i64r": as_f32(i64.real), "i64i": as_f32(i64.imag),
        "rev64": as_f32(rev64), "rev128": as_f32(rev128),
    }


def _dot(a, bm):
    return jax.lax.dot_general(
        a, bm, (((1,), (0,)), ((), ())), precision=HI,
        preferred_element_type=jnp.float32)


def _proj_body(x_ref, wt_ref, b_ref, oa_ref, ob_ref):
    res = _dot(x_ref[...], wt_ref[...]) + b_ref[...]
    ra = res[:, :DP]
    rb = res[:, DP:]
    oa_ref[0, 0] = ra
    oa_ref[0, 1] = ra
    ob_ref[0, 0] = rb
    ob_ref[0, 1] = rb


def _projection(x, W, b):
    x2 = x.reshape(B * T, D)
    wt = W.T
    b2 = b.reshape(1, D)
    rows = 1024
    nb = T // rows
    qa, qb = pl.pallas_call(
        _proj_body,
        grid=(B, nb),
        in_specs=[
            pl.BlockSpec(
                (rows, D), lambda bb, i, _nb=nb: (bb * _nb + i, 0)),
            pl.BlockSpec((D, D), lambda bb, i: (0, 0)),
            pl.BlockSpec((1, D), lambda bb, i: (0, 0)),
        ],
        out_specs=[
            pl.BlockSpec((1, 2, rows, DP), lambda bb, i: (bb, 0, i, 0)),
            pl.BlockSpec((1, 2, rows, DP), lambda bb, i: (bb, 0, i, 0)),
        ],
        out_shape=[
            jax.ShapeDtypeStruct((B, 2, T, DP), jnp.float32),
            jax.ShapeDtypeStruct((B, 2, T, DP), jnp.float32),
        ],
    )(x2, wt, b2)
    return qa.reshape(B, 2 * T, DP), qb.reshape(B, 2 * T, DP)


def _fft1_body(zr_ref, zi_ref, e64r, e64i, yr_ref, yi_ref):
    zr = zr_ref[0]
    zi = zi_ref[0]
    yr_ref[0, 0] = _dot(e64r[...], zr) - _dot(e64i[...], zi)
    yi_ref[0, 0] = _dot(e64r[...], zi) + _dot(e64i[...], zr)


def _fft_stage1(qa_ext, qb_ext, c):
    za = qa_ext.reshape(B, 2 * N1, N2 * DP)
    zb = qb_ext.reshape(B, 2 * N1, N2 * DP)
    return pl.pallas_call(
        _fft1_body,
        grid=(B, _NK),
        in_specs=[
            pl.BlockSpec((1, N1, _MC), lambda b, k: (b, 0, k)),
            pl.BlockSpec((1, N1, _MC), lambda b, k: (b, 0, k)),
            pl.BlockSpec((N1, N1), lambda b, k: (0, 0)),
            pl.BlockSpec((N1, N1), lambda b, k: (0, 0)),
        ],
        out_specs=[
            pl.BlockSpec((1, 1, N1, _MC), lambda b, k: (b, k, 0, 0)),
            pl.BlockSpec((1, 1, N1, _MC), lambda b, k: (b, k, 0, 0)),
        ],
        out_shape=[
            jax.ShapeDtypeStruct((B, _NK, N1, _MC), jnp.float32),
            jax.ShapeDtypeStruct((B, _NK, N1, _MC), jnp.float32),
        ],
    )(za, zb, c["e64r"], c["e64i"])


def _fft2_body(yr_ref, yi_ref, twr, twi, f128r, f128i, s_ref):
    g = pl.program_id(1)
    for u in range(2):
        yr = yr_ref[0, :, u].reshape(N2, DP)
        yi = yi_ref[0, :, u].reshape(N2, DP)
        wr = twr[0, u]
        wi = twi[0, u]
        zr = yr * wr - yi * wi
        zi = yr * wi + yi * wr
        xr = _dot(f128r[...], zr) - _dot(f128i[...], zi)
        xi = _dot(f128r[...], zi) + _dot(f128i[...], zr)
        s_ref[0, 2 * g + u, :] = jnp.sum(xr * xr + xi * xi, axis=1)


def _fft_stage2(yr4, yi4, c):
    yr5 = yr4.reshape(B, _NK, N1, _TC, DP)
    yi5 = yi4.reshape(B, _NK, N1, _TC, DP)
    twr = c["twr"].reshape(N1 // 2, 2, N2, 1)
    twi = c["twi"].reshape(N1 // 2, 2, N2, 1)
    return pl.pallas_call(
        _fft2_body,
        grid=(B, N1 // 2),
        in_specs=[
            pl.BlockSpec((1, _NK, 2, _TC, DP),
                         lambda b, g: (b, 0, g, 0, 0)),
            pl.BlockSpec((1, _NK, 2, _TC, DP),
                         lambda b, g: (b, 0, g, 0, 0)),
            pl.BlockSpec((1, 2, N2, 1), lambda b, g: (g, 0, 0, 0)),
            pl.BlockSpec((1, 2, N2, 1), lambda b, g: (g, 0, 0, 0)),
            pl.BlockSpec((N2, N2), lambda b, g: (0, 0)),
            pl.BlockSpec((N2, N2), lambda b, g: (0, 0)),
        ],
        out_specs=pl.BlockSpec((1, N1, N2), lambda b, g: (b, 0, 0)),
        out_shape=jax.ShapeDtypeStruct((B, N1, N2), jnp.float32),
    )(yr5, yi5, twr, twi, c["f128r"], c["f128i"])


def _cmul(ar, ai, br, bi):
    return ar * br - ai * bi, ar * bi + ai * br


def _topk_body(s_ref, icr, ici, itwr, itwi, i64r, i64i, r64, r128, d_ref):
    s = s_ref[0]
    gr = jnp.dot(s, icr[...], precision=HI)
    gi = jnp.dot(s, ici[...], precision=HI)
    hr, hi = _cmul(gr, gi, itwr[...], itwi[...])
    corr = (jnp.dot(i64r[...], hr, precision=HI)
            - jnp.dot(i64i[...], hi, precision=HI))
    m1 = jnp.dot(jnp.dot(r64[...], corr, precision=HI), r128[...],
                 precision=HI)
    r = jnp.roll(m1, 1, axis=1)
    colidx = jax.lax.broadcasted_iota(jnp.int32, (N1, N2), 1)
    mirror = jnp.where(colidx == 0, jnp.roll(r, 1, axis=0), r)
    flat = (jax.lax.broadcasted_iota(jnp.int32, (N1, N2), 0) * N2
            + jax.lax.broadcasted_iota(jnp.int32, (N1, N2), 1))
    neg = jnp.float32(-jnp.inf)
    c = corr + mirror
    for i in range(TOPK):
        m = jnp.max(c)
        idx = jnp.min(jnp.where(c == m, flat, T))
        d_ref[0, 0, i] = idx
        c = jnp.where(flat == idx, neg, c)
    for i in range(TOPK, 16):
        d_ref[0, 0, i] = 0


def _top_delays(s, c):
    return pl.pallas_call(
        _topk_body,
        grid=(B,),
        in_specs=[
            pl.BlockSpec((1, N1, N2), lambda b: (b, 0, 0)),
            pl.BlockSpec((N2, N2), lambda b: (0, 0)),
            pl.BlockSpec((N2, N2), lambda b: (0, 0)),
            pl.BlockSpec((N1, N2), lambda b: (0, 0)),
            pl.BlockSpec((N1, N2), lambda b: (0, 0)),
            pl.BlockSpec((N1, N1), lambda b: (0, 0)),
            pl.BlockSpec((N1, N1), lambda b: (0, 0)),
            pl.BlockSpec((N1, N1), lambda b: (0, 0)),
            pl.BlockSpec((N2, N2), lambda b: (0, 0)),
        ],
        out_specs=pl.BlockSpec(
            (1, 1, 16), lambda b: (b, 0, 0), memory_space=pltpu.SMEM),
        out_shape=jax.ShapeDtypeStruct((B, 1, 16), jnp.int32),
    )(s, c["icr"], c["ici"], c["itwr"], c["itwi"], c["i64r"], c["i64i"],
      c["rev64"], c["rev128"])


def _combine_body(d_ref, qa_ref, qb_ref, v_ref, o_ref):
    b = pl.program_id(0)
    h = pl.program_id(2)
    inv = jnp.float32(1.0 / TOPK)

    @pl.when(h == 0)
    def _():
        acc = qa_ref[0, pl.ds(d_ref[b, 0], T), :]
        for i in range(1, TOPK):
            acc = acc + qa_ref[0, pl.ds(d_ref[b, i], T), :]
        o_ref[0] = v_ref[0] + acc * inv

    @pl.when(h == 1)
    def _():
        acc = qb_ref[0, pl.ds(d_ref[b, 0], T), :]
        for i in range(1, TOPK):
            acc = acc + qb_ref[0, pl.ds(d_ref[b, i], T), :]
        o_ref[0] = v_ref[0] + acc * inv


def _combine(delays, qa_ext, qb_ext, x):
    dc = 128
    nj = DP // dc
    return pl.pallas_call(
        _combine_body,
        grid=(B, nj, 2),
        in_specs=[
            pl.BlockSpec((B, 16), lambda b, j, h: (0, 0),
                         memory_space=pltpu.SMEM),
            pl.BlockSpec((1, 2 * T, dc), lambda b, j, h: (b, 0, j)),
            pl.BlockSpec((1, 2 * T, dc), lambda b, j, h: (b, 0, j)),
            pl.BlockSpec((1, T, dc),
                         lambda b, j, h, _nj=nj: (b, 0, h * _nj + j)),
        ],
        out_specs=pl.BlockSpec(
            (1, T, dc), lambda b, j, h, _nj=nj: (b, 0, h * _nj + j)),
        out_shape=jax.ShapeDtypeStruct((B, T, D), jnp.float32),
    )(delays, qa_ext, qb_ext, x)


def kernel(x, W, b):
    c = _consts()
    qa_ext, qb_ext = _projection(x, W, b)
    yr4, yi4 = _fft_stage1(qa_ext, qb_ext, c)
    s = _fft_stage2(yr4, yi4, c)
    delays = _top_delays(s, c).reshape(B, 16)
    return _combine(delays, qa_ext, qb_ext, x)
